# Initial kernel scaffold; baseline (speedup 1.0000x reference)
import jax
import jax.numpy as jnp
from jax.experimental import pallas as pl


def kernel(z, emb_w):
    raise NotImplementedError("write your pallas kernel here")


# speedup vs baseline: 6.7256x; 6.7256x over previous
import jax
import jax.numpy as jnp
from jax.experimental import pallas as pl

N_E = 8192
E_DIM = 32
N_TOK = 8192
BETA = 0.25
BT = 256
NT = N_TOK // BT


def _vq_body(idx_ref, z_ref, emb_ref, enc_ref, zq_ref, cnt_ref, loss_ref,
             perp_ref):
    i = pl.program_id(0)
    idx = idx_ref[0, 0, :]
    iota = jax.lax.broadcasted_iota(jnp.int32, (BT, N_E), 1)
    oh = (iota == idx[:, None]).astype(jnp.float32)
    enc_ref[...] = oh
    emb = emb_ref[...]
    zq = jnp.dot(oh, emb, preferred_element_type=jnp.float32)
    zq_ref[...] = zq
    z = z_ref[...]
    part = jnp.sum(oh, axis=0)
    sq = jnp.sum((zq - z) ** 2)

    @pl.when(i == 0)
    def _init():
        cnt_ref[...] = jnp.zeros_like(cnt_ref)
        loss_ref[...] = jnp.zeros_like(loss_ref)
        perp_ref[...] = jnp.zeros_like(perp_ref)

    cnt_ref[...] += part[None, :]
    loss_ref[...] += jnp.full((1, 1), sq, jnp.float32)

    @pl.when(i == NT - 1)
    def _fini():
        loss_ref[...] = loss_ref[...] * ((1.0 + BETA) / (N_TOK * E_DIM))
        em = cnt_ref[...] * (1.0 / N_TOK)
        ent = jnp.sum(em * jnp.log(em + 1e-10))
        perp_ref[...] = jnp.full((1, 1), jnp.exp(-ent), jnp.float32)


_vq_call = pl.pallas_call(
    _vq_body,
    grid=(NT,),
    in_specs=[
        pl.BlockSpec((1, 1, BT), lambda i: (i, 0, 0)),
        pl.BlockSpec((BT, E_DIM), lambda i: (i, 0)),
        pl.BlockSpec((N_E, E_DIM), lambda i: (0, 0)),
    ],
    out_specs=[
        pl.BlockSpec((BT, N_E), lambda i: (i, 0)),
        pl.BlockSpec((BT, E_DIM), lambda i: (i, 0)),
        pl.BlockSpec((1, N_E), lambda i: (0, 0)),
        pl.BlockSpec((1, 1), lambda i: (0, 0)),
        pl.BlockSpec((1, 1), lambda i: (0, 0)),
    ],
    out_shape=[
        jax.ShapeDtypeStruct((N_TOK, N_E), jnp.float32),
        jax.ShapeDtypeStruct((N_TOK, E_DIM), jnp.float32),
        jax.ShapeDtypeStruct((1, N_E), jnp.float32),
        jax.ShapeDtypeStruct((1, 1), jnp.float32),
        jax.ShapeDtypeStruct((1, 1), jnp.float32),
    ],
)


def kernel(z, emb_w):
    bt, ch, h, w = z.shape
    z_p = jnp.transpose(z, (0, 2, 3, 1))
    z_flat = z_p.reshape(-1, E_DIM)
    d = (jnp.sum(z_flat ** 2, axis=1, keepdims=True)
         + jnp.sum(emb_w ** 2, axis=1)
         - 2.0 * jnp.matmul(z_flat, emb_w.T))
    min_idx = jnp.argmin(d, axis=1)
    zb, eb, ib = jax.lax.optimization_barrier((z, emb_w, min_idx))
    z_flat2 = jnp.transpose(zb, (0, 2, 3, 1)).reshape(-1, E_DIM)
    enc, zq_flat, _cnt, loss, perp = _vq_call(
        ib.reshape(NT, 1, BT), z_flat2, eb)
    z_q_out = jnp.transpose(zq_flat.reshape(bt, h, w, E_DIM), (0, 3, 1, 2))
    idx_out = min_idx.reshape(bt, h, w)
    return (loss[0, 0], z_q_out, perp[0, 0], enc, idx_out)
